# baseline (speedup 1.0000x reference)
import functools

import jax
import jax.numpy as jnp
from jax import lax
from jax.experimental import pallas as pl
from jax.experimental.pallas import tpu as pltpu
from jax.experimental.pallas import tpu_sc as plsc

EPS = 1e-12

_NUM_CORES = 2
_NUM_SUBCORES = 16
_NW = _NUM_CORES * _NUM_SUBCORES

_CHUNK = 64

_PIPE_CHUNKS = (512, 1024, 1024, 1024, 512)

_BS = 512


def _sc_gather(table, idx):
    tok, h = idx.shape[0], table.shape[1]
    per_w = tok // _NW
    chunk = min(_CHUNK, per_w)
    n_chunks = per_w // chunk
    assert per_w % chunk == 0

    mesh = plsc.VectorSubcoreMesh(
        core_axis_name="c", subcore_axis_name="s",
        num_cores=_NUM_CORES, num_subcores=_NUM_SUBCORES)

    @functools.partial(
        pl.kernel,
        mesh=mesh,
        out_type=jax.ShapeDtypeStruct((tok, h), jnp.float32),
        scratch_types=[
            pltpu.VMEM((per_w,), jnp.int32),
            pltpu.VMEM((chunk, h), jnp.float32),
            pltpu.VMEM((chunk, h), jnp.float32),
            pltpu.SemaphoreType.DMA,
            pltpu.SemaphoreType.DMA,
        ],
    )
    def gather_kernel(table_hbm, idx_hbm, out_hbm, idx_v, buf0, buf1, sem0, sem1):
        wid = lax.axis_index("s") * _NUM_CORES + lax.axis_index("c")
        base = wid * per_w
        pltpu.sync_copy(idx_hbm.at[pl.ds(base, per_w)], idx_v)

        bufs = (buf0, buf1)
        sems = (sem0, sem1)
        copies = [None, None]
        copies[0] = pltpu.async_copy(
            table_hbm.at[idx_v.at[pl.ds(0, chunk)]], bufs[0], sems[0])
        for c in range(1, n_chunks):
            copies[c % 2] = pltpu.async_copy(
                table_hbm.at[idx_v.at[pl.ds(c * chunk, chunk)]],
                bufs[c % 2], sems[c % 2])
            copies[(c - 1) % 2].wait()
            pltpu.sync_copy(
                bufs[(c - 1) % 2],
                out_hbm.at[pl.ds(base + (c - 1) * chunk, chunk)])
        copies[(n_chunks - 1) % 2].wait()
        pltpu.sync_copy(
            bufs[(n_chunks - 1) % 2],
            out_hbm.at[pl.ds(base + (n_chunks - 1) * chunk, chunk)])

    return gather_kernel(table, idx)


def _ln_body(x_ref, pos_ref, g_ref, b_ref, o_ref):
    xv = x_ref[0] + pos_ref[...]
    mean = jnp.mean(xv, axis=-1, keepdims=True)
    xc = xv - mean
    var = jnp.mean(xc * xc, axis=-1, keepdims=True)
    inv = lax.rsqrt(var + EPS)
    o_ref[0] = (xc * inv) * g_ref[...] + b_ref[...]


def _add_ln_chunk(x_c, pos, gamma, beta, out_buf, seq_start, out_seq):
    b, sc, h = x_c.shape
    nblk = sc // _BS
    off = seq_start // _BS
    grid = (nblk, b)

    in_specs = [
        pl.BlockSpec((1, _BS, h), lambda i, j: (j, i, 0)),
        pl.BlockSpec((_BS, h), lambda i, j: (off + i, 0)),
        pl.BlockSpec((1, h), lambda i, j: (0, 0)),
        pl.BlockSpec((1, h), lambda i, j: (0, 0)),
    ]
    args = [x_c, pos, gamma, beta]
    aliases = {}
    body = _ln_body
    if out_buf is not None:
        in_specs.append(pl.BlockSpec(memory_space=pl.ANY))
        args.append(out_buf)
        aliases = {4: 0}
        body = lambda x, p, g, bt, _, o: _ln_body(x, p, g, bt, o)

    return pl.pallas_call(
        body,
        grid=grid,
        in_specs=in_specs,
        out_specs=pl.BlockSpec((1, _BS, h), lambda i, j: (j, off + i, 0)),
        out_shape=jax.ShapeDtypeStruct((b, out_seq, h), jnp.float32),
        input_output_aliases=aliases,
    )(*args)


def kernel(input_ids, word_table, pos_table, ln_gamma, ln_beta):
    b, s = input_ids.shape
    h = word_table.shape[1]
    assert sum(_PIPE_CHUNKS) == s
    gamma2 = ln_gamma.reshape(1, h)
    beta2 = ln_beta.reshape(1, h)
    ids32 = input_ids.astype(jnp.int32)
    pos = pos_table[:s]

    starts = [sum(_PIPE_CHUNKS[:c]) for c in range(len(_PIPE_CHUNKS))]
    gathered = [
        _sc_gather(word_table,
                   lax.slice(ids32, (0, st), (b, st + sc)).reshape(-1))
        for st, sc in zip(starts, _PIPE_CHUNKS)
    ]
    out = None
    for g, st, sc in zip(gathered, starts, _PIPE_CHUNKS):
        out = _add_ln_chunk(g.reshape(b, sc, h), pos, gamma2, beta2, out, st, s)
    return out


# speedup vs baseline: 1.0632x; 1.0632x over previous
import functools

import jax
import jax.numpy as jnp
from jax import lax
from jax.experimental import pallas as pl
from jax.experimental.pallas import tpu as pltpu
from jax.experimental.pallas import tpu_sc as plsc

EPS = 1e-12

_NUM_CORES = 2
_NUM_SUBCORES = 16
_NW = _NUM_CORES * _NUM_SUBCORES

_CHUNK = 64

_PIPE_CHUNKS = (1024, 1024, 1024, 1024)

_BS = 1024


def _sc_gather(table, idx):
    tok, h = idx.shape[0], table.shape[1]
    per_w = tok // _NW
    chunk = min(_CHUNK, per_w)
    n_chunks = per_w // chunk
    assert per_w % chunk == 0

    mesh = plsc.VectorSubcoreMesh(
        core_axis_name="c", subcore_axis_name="s",
        num_cores=_NUM_CORES, num_subcores=_NUM_SUBCORES)

    @functools.partial(
        pl.kernel,
        mesh=mesh,
        out_type=jax.ShapeDtypeStruct((tok, h), jnp.float32),
        scratch_types=[
            pltpu.VMEM((per_w,), jnp.int32),
            pltpu.VMEM((chunk, h), jnp.float32),
            pltpu.VMEM((chunk, h), jnp.float32),
            pltpu.SemaphoreType.DMA,
            pltpu.SemaphoreType.DMA,
        ],
    )
    def gather_kernel(table_hbm, idx_hbm, out_hbm, idx_v, buf0, buf1, sem0, sem1):
        wid = lax.axis_index("s") * _NUM_CORES + lax.axis_index("c")
        base = wid * per_w
        pltpu.sync_copy(idx_hbm.at[pl.ds(base, per_w)], idx_v)

        bufs = (buf0, buf1)
        sems = (sem0, sem1)
        copies = [None, None]
        copies[0] = pltpu.async_copy(
            table_hbm.at[idx_v.at[pl.ds(0, chunk)]], bufs[0], sems[0])
        for c in range(1, n_chunks):
            copies[c % 2] = pltpu.async_copy(
                table_hbm.at[idx_v.at[pl.ds(c * chunk, chunk)]],
                bufs[c % 2], sems[c % 2])
            copies[(c - 1) % 2].wait()
            pltpu.sync_copy(
                bufs[(c - 1) % 2],
                out_hbm.at[pl.ds(base + (c - 1) * chunk, chunk)])
        copies[(n_chunks - 1) % 2].wait()
        pltpu.sync_copy(
            bufs[(n_chunks - 1) % 2],
            out_hbm.at[pl.ds(base + (n_chunks - 1) * chunk, chunk)])

    return gather_kernel(table, idx)


def _ln_body(x_ref, pos_ref, g_ref, b_ref, o_ref):
    xv = x_ref[0] + pos_ref[...]
    mean = jnp.mean(xv, axis=-1, keepdims=True)
    xc = xv - mean
    var = jnp.mean(xc * xc, axis=-1, keepdims=True)
    inv = lax.rsqrt(var + EPS)
    o_ref[0] = (xc * inv) * g_ref[...] + b_ref[...]


def _add_ln_chunk(x_c, pos, gamma, beta, out_buf, seq_start, out_seq):
    b, sc, h = x_c.shape
    nblk = sc // _BS
    off = seq_start // _BS
    grid = (nblk, b)

    in_specs = [
        pl.BlockSpec((1, _BS, h), lambda i, j: (j, i, 0)),
        pl.BlockSpec((_BS, h), lambda i, j: (off + i, 0)),
        pl.BlockSpec((1, h), lambda i, j: (0, 0)),
        pl.BlockSpec((1, h), lambda i, j: (0, 0)),
    ]
    args = [x_c, pos, gamma, beta]
    aliases = {}
    body = _ln_body
    if out_buf is not None:
        in_specs.append(pl.BlockSpec(memory_space=pl.ANY))
        args.append(out_buf)
        aliases = {4: 0}
        body = lambda x, p, g, bt, _, o: _ln_body(x, p, g, bt, o)

    return pl.pallas_call(
        body,
        grid=grid,
        in_specs=in_specs,
        out_specs=pl.BlockSpec((1, _BS, h), lambda i, j: (j, off + i, 0)),
        out_shape=jax.ShapeDtypeStruct((b, out_seq, h), jnp.float32),
        input_output_aliases=aliases,
    )(*args)


def kernel(input_ids, word_table, pos_table, ln_gamma, ln_beta):
    b, s = input_ids.shape
    h = word_table.shape[1]
    assert sum(_PIPE_CHUNKS) == s
    gamma2 = ln_gamma.reshape(1, h)
    beta2 = ln_beta.reshape(1, h)
    ids32 = input_ids.astype(jnp.int32)
    pos = pos_table[:s]

    starts = [sum(_PIPE_CHUNKS[:c]) for c in range(len(_PIPE_CHUNKS))]
    gathered = [
        _sc_gather(word_table,
                   lax.slice(ids32, (0, st), (b, st + sc)).reshape(-1))
        for st, sc in zip(starts, _PIPE_CHUNKS)
    ]
    out = None
    for g, st, sc in zip(gathered, starts, _PIPE_CHUNKS):
        out = _add_ln_chunk(g.reshape(b, sc, h), pos, gamma2, beta2, out, st, s)
    return out
